# initial kernel scaffold (unmeasured)
import jax
import jax.numpy as jnp
from jax import lax
from jax.experimental import pallas as pl
from jax.experimental.pallas import tpu as pltpu


def kernel(
    x,
):
    def body(*refs):
        pass

    out_shape = jax.ShapeDtypeStruct(..., jnp.float32)
    return pl.pallas_call(body, out_shape=out_shape)(...)


# baseline (device time: 1099874 ns/iter reference)
import jax
import jax.numpy as jnp
from jax import lax
from jax.experimental import pallas as pl
from jax.experimental.pallas import tpu as pltpu


def kernel(x):
    m_per, n = x.shape
    xb = x.astype(jnp.bfloat16)

    def body(x_ref, out_ref, local_sem, send_sem, recv_sem):
        my_x = lax.axis_index("x")
        my_y = lax.axis_index("y")
        my_z = lax.axis_index("z")
        other_z = 1 - my_z

        barrier_sem = pltpu.get_barrier_semaphore()
        pl.semaphore_signal(
            barrier_sem,
            inc=1,
            device_id=(my_x, my_y, other_z),
            device_id_type=pl.DeviceIdType.MESH,
        )
        pl.semaphore_wait(barrier_sem, 1)

        local = pltpu.make_async_copy(
            x_ref, out_ref.at[pl.ds(my_z * m_per, m_per), :], local_sem
        )
        local.start()

        rdma = pltpu.make_async_remote_copy(
            src_ref=x_ref,
            dst_ref=out_ref.at[pl.ds(my_z * m_per, m_per), :],
            send_sem=send_sem,
            recv_sem=recv_sem,
            device_id=(my_x, my_y, other_z),
            device_id_type=pl.DeviceIdType.MESH,
        )
        rdma.start()
        local.wait()
        rdma.wait()

    return pl.pallas_call(
        body,
        out_shape=jax.ShapeDtypeStruct((2 * m_per, n), jnp.bfloat16),
        in_specs=[pl.BlockSpec(memory_space=pl.ANY)],
        out_specs=pl.BlockSpec(memory_space=pl.ANY),
        scratch_shapes=[
            pltpu.SemaphoreType.DMA,
            pltpu.SemaphoreType.DMA,
            pltpu.SemaphoreType.DMA,
        ],
        compiler_params=pltpu.CompilerParams(collective_id=0),
    )(xb)


# device time: 265233 ns/iter; 4.1468x vs baseline; 4.1468x over previous
import jax
import jax.numpy as jnp
from jax import lax
from jax.experimental import pallas as pl
from jax.experimental.pallas import tpu as pltpu

C = 8


def kernel(x):
    m, n = x.shape
    h = m // 2
    c = h // C

    def body(x_ref, out_ref, vx, vc, load_sems, store_sems,
             z_send, z_recv, xf_send, xf_recv):
        my_x = lax.axis_index("x")
        my_y = lax.axis_index("y")
        my_z = lax.axis_index("z")
        z_peer = (my_x, my_y, 1 - my_z)
        x_peer = (1 - my_x, my_y, my_z)

        barrier_sem = pltpu.get_barrier_semaphore()
        for peer in (z_peer, x_peer):
            pl.semaphore_signal(
                barrier_sem, inc=1, device_id=peer,
                device_id_type=pl.DeviceIdType.MESH,
            )
        pl.semaphore_wait(barrier_sem, 2)

        my_base = my_z * m
        oth_base = (1 - my_z) * m
        send_off = my_x * h
        keep_off = h - send_off

        offs = [send_off + k * c for k in range(C)] + [
            keep_off + k * c for k in range(C)
        ]

        loads = []
        for j in range(2 * C):
            loads.append(
                pltpu.make_async_copy(
                    x_ref.at[pl.ds(offs[j], c), :], vx.at[j % 2],
                    load_sems.at[j % 2],
                )
            )
        stores = [None] * (2 * C)
        store_waited = [False] * (2 * C)
        z_rdmas = [None] * C
        loads[0].start()
        for j in range(2 * C):
            if j + 1 < 2 * C:
                loads[j + 1].start()
            loads[j].wait()
            if j >= 2 and not store_waited[j - 2]:
                stores[j - 2].wait()
                store_waited[j - 2] = True
            vc[j % 2] = vx[j % 2].astype(jnp.bfloat16)
            st = pltpu.make_async_copy(
                vc.at[j % 2], out_ref.at[pl.ds(my_base + offs[j], c), :],
                store_sems.at[j],
            )
            st.start()
            stores[j] = st
            if j < C:
                st.wait()
                store_waited[j] = True
                rd = pltpu.make_async_remote_copy(
                    src_ref=out_ref.at[pl.ds(my_base + offs[j], c), :],
                    dst_ref=out_ref.at[pl.ds(my_base + offs[j], c), :],
                    send_sem=z_send.at[j], recv_sem=z_recv.at[j],
                    device_id=z_peer, device_id_type=pl.DeviceIdType.MESH,
                )
                rd.start()
                z_rdmas[j] = rd
        for j in range(2 * C):
            if not store_waited[j]:
                stores[j].wait()
                store_waited[j] = True

        x_rdmas = []
        for k in range(C):
            z_rdmas[k].wait_recv()
            row = oth_base + send_off + k * c
            fw = pltpu.make_async_remote_copy(
                src_ref=out_ref.at[pl.ds(row, c), :],
                dst_ref=out_ref.at[pl.ds(row, c), :],
                send_sem=xf_send.at[k], recv_sem=xf_recv.at[k],
                device_id=x_peer, device_id_type=pl.DeviceIdType.MESH,
            )
            fw.start()
            x_rdmas.append(fw)
        for k in range(C):
            x_rdmas[k].wait_recv()
        for k in range(C):
            z_rdmas[k].wait_send()
            x_rdmas[k].wait_send()

    return pl.pallas_call(
        body,
        out_shape=jax.ShapeDtypeStruct((2 * m, n), jnp.bfloat16),
        in_specs=[pl.BlockSpec(memory_space=pl.ANY)],
        out_specs=pl.BlockSpec(memory_space=pl.ANY),
        scratch_shapes=[
            pltpu.VMEM((2, c, n), jnp.float32),
            pltpu.VMEM((2, c, n), jnp.bfloat16),
            pltpu.SemaphoreType.DMA((2,)),
            pltpu.SemaphoreType.DMA((2 * C,)),
            pltpu.SemaphoreType.DMA((C,)),
            pltpu.SemaphoreType.DMA((C,)),
            pltpu.SemaphoreType.DMA((C,)),
            pltpu.SemaphoreType.DMA((C,)),
        ],
        compiler_params=pltpu.CompilerParams(collective_id=0),
    )(x)


# device time: 250477 ns/iter; 4.3911x vs baseline; 1.0589x over previous
import jax
import jax.numpy as jnp
from jax import lax
from jax.experimental import pallas as pl
from jax.experimental.pallas import tpu as pltpu

C = 16


def kernel(x):
    m, n = x.shape
    h = m // 2
    c = h // C

    def body(x_ref, out_ref, vx, vc, vk, load_sems, store_sems,
             z_send, z_recv, xf_send, xf_recv):
        my_x = lax.axis_index("x")
        my_y = lax.axis_index("y")
        my_z = lax.axis_index("z")
        z_peer = (my_x, my_y, 1 - my_z)
        x_peer = (1 - my_x, my_y, my_z)

        barrier_sem = pltpu.get_barrier_semaphore()
        for peer in (z_peer, x_peer):
            pl.semaphore_signal(
                barrier_sem, inc=1, device_id=peer,
                device_id_type=pl.DeviceIdType.MESH,
            )
        pl.semaphore_wait(barrier_sem, 2)

        my_base = my_z * m
        oth_base = (1 - my_z) * m
        send_off = my_x * h
        keep_off = h - send_off

        offs = [send_off + k * c for k in range(C)] + [
            keep_off + k * c for k in range(C)
        ]

        loads = []
        for j in range(2 * C):
            loads.append(
                pltpu.make_async_copy(
                    x_ref.at[pl.ds(offs[j], c), :], vx.at[j % 2],
                    load_sems.at[j % 2],
                )
            )
        stores = [None] * (2 * C)
        store_waited = [False] * (2 * C)
        z_rdmas = [None] * C
        x_rdmas = [None] * C
        loads[0].start()
        for j in range(2 * C):
            if j + 1 < 2 * C:
                loads[j + 1].start()
            loads[j].wait()
            if j < C:
                cast_buf = vc.at[j]
                vc[j] = vx[j % 2].astype(jnp.bfloat16)
            else:
                if j >= C + 2 and not store_waited[j - 2]:
                    stores[j - 2].wait()
                    store_waited[j - 2] = True
                cast_buf = vk.at[j % 2]
                vk[j % 2] = vx[j % 2].astype(jnp.bfloat16)
            st = pltpu.make_async_copy(
                cast_buf, out_ref.at[pl.ds(my_base + offs[j], c), :],
                store_sems.at[j],
            )
            st.start()
            stores[j] = st
            if j < C:
                rd = pltpu.make_async_remote_copy(
                    src_ref=vc.at[j],
                    dst_ref=out_ref.at[pl.ds(my_base + offs[j], c), :],
                    send_sem=z_send.at[j], recv_sem=z_recv.at[j],
                    device_id=z_peer, device_id_type=pl.DeviceIdType.MESH,
                )
                rd.start()
                z_rdmas[j] = rd
            else:
                k = j - C
                z_rdmas[k].wait_recv()
                row = oth_base + send_off + k * c
                fw = pltpu.make_async_remote_copy(
                    src_ref=out_ref.at[pl.ds(row, c), :],
                    dst_ref=out_ref.at[pl.ds(row, c), :],
                    send_sem=xf_send.at[k], recv_sem=xf_recv.at[k],
                    device_id=x_peer, device_id_type=pl.DeviceIdType.MESH,
                )
                fw.start()
                x_rdmas[k] = fw
        for k in range(C):
            x_rdmas[k].wait_recv()
        for j in range(2 * C):
            if not store_waited[j]:
                stores[j].wait()
        for k in range(C):
            z_rdmas[k].wait_send()
            x_rdmas[k].wait_send()

    return pl.pallas_call(
        body,
        out_shape=jax.ShapeDtypeStruct((2 * m, n), jnp.bfloat16),
        in_specs=[pl.BlockSpec(memory_space=pl.ANY)],
        out_specs=pl.BlockSpec(memory_space=pl.ANY),
        scratch_shapes=[
            pltpu.VMEM((2, c, n), jnp.float32),
            pltpu.VMEM((C, c, n), jnp.bfloat16),
            pltpu.VMEM((2, c, n), jnp.bfloat16),
            pltpu.SemaphoreType.DMA((2,)),
            pltpu.SemaphoreType.DMA((2 * C,)),
            pltpu.SemaphoreType.DMA((C,)),
            pltpu.SemaphoreType.DMA((C,)),
            pltpu.SemaphoreType.DMA((C,)),
            pltpu.SemaphoreType.DMA((C,)),
        ],
        compiler_params=pltpu.CompilerParams(collective_id=0),
    )(x)


# device time: 244430 ns/iter; 4.4998x vs baseline; 1.0247x over previous
import jax
import jax.numpy as jnp
from jax import lax
from jax.experimental import pallas as pl
from jax.experimental.pallas import tpu as pltpu

C = 16
NSLOTS = 6


def kernel(x):
    m, n = x.shape
    h = m // 2
    c = h // C

    def body(x_ref, out_ref, vx, vc, vk, load_sems, store_sems,
             z_send, z_recv, xf_send, xf_recv):
        my_x = lax.axis_index("x")
        my_y = lax.axis_index("y")
        my_z = lax.axis_index("z")
        z_peer = (my_x, my_y, 1 - my_z)
        x_peer = (1 - my_x, my_y, my_z)

        barrier_sem = pltpu.get_barrier_semaphore()
        for peer in (z_peer, x_peer):
            pl.semaphore_signal(
                barrier_sem, inc=1, device_id=peer,
                device_id_type=pl.DeviceIdType.MESH,
            )
        pl.semaphore_wait(barrier_sem, 2)

        my_base = my_z * m
        oth_base = (1 - my_z) * m
        send_off = my_x * h
        keep_off = h - send_off

        plan = [(0, None, None), (1, None, None)]
        for t in range(C - 2):
            plan.append((2 + t, t, t))
        plan.append((None, C - 2, C - 2))
        plan.append((None, C - 1, C - 1))

        load_items = []
        for s_j, k_j, _ in plan:
            if s_j is not None:
                load_items.append(("s", s_j))
            if k_j is not None:
                load_items.append(("k", k_j))
        lmap = {}
        loads = []
        for li, (kind, idx) in enumerate(load_items):
            off = (send_off if kind == "s" else keep_off) + idx * c
            loads.append(
                pltpu.make_async_copy(
                    x_ref.at[pl.ds(off, c), :], vx.at[li % NSLOTS],
                    load_sems.at[li % NSLOTS],
                )
            )
            lmap[(kind, idx)] = li

        state = {"started": 0, "casted": 0}

        def pump_loads():
            while (state["started"] < len(loads)
                   and state["started"] - state["casted"] < NSLOTS - 1):
                loads[state["started"]].start()
                state["started"] += 1

        stores = {}
        store_waited = set()
        z_rdmas = [None] * C
        x_rdmas = [None] * C
        pump_loads()

        for s_j, k_j, f_k in plan:
            for kind, idx in (("s", s_j), ("k", k_j)):
                if idx is None:
                    continue
                li = lmap[(kind, idx)]
                loads[li].wait()
                state["casted"] += 1
                pump_loads()
                if kind == "s":
                    vc[idx] = vx[li % NSLOTS].astype(jnp.bfloat16)
                    buf = vc.at[idx]
                    row = my_base + send_off + idx * c
                    sidx = idx
                else:
                    if idx >= 2 and ("k", idx - 2) not in store_waited:
                        stores[("k", idx - 2)].wait()
                        store_waited.add(("k", idx - 2))
                    vk[idx % 2] = vx[li % NSLOTS].astype(jnp.bfloat16)
                    buf = vk.at[idx % 2]
                    row = my_base + keep_off + idx * c
                    sidx = C + idx
                st = pltpu.make_async_copy(
                    buf, out_ref.at[pl.ds(row, c), :], store_sems.at[sidx]
                )
                st.start()
                stores[(kind, idx)] = st
                if kind == "s":
                    rd = pltpu.make_async_remote_copy(
                        src_ref=vc.at[idx],
                        dst_ref=out_ref.at[pl.ds(row, c), :],
                        send_sem=z_send.at[idx], recv_sem=z_recv.at[idx],
                        device_id=z_peer, device_id_type=pl.DeviceIdType.MESH,
                    )
                    rd.start()
                    z_rdmas[idx] = rd
            if f_k is not None:
                z_rdmas[f_k].wait_recv()
                row = oth_base + send_off + f_k * c
                fw = pltpu.make_async_remote_copy(
                    src_ref=out_ref.at[pl.ds(row, c), :],
                    dst_ref=out_ref.at[pl.ds(row, c), :],
                    send_sem=xf_send.at[f_k], recv_sem=xf_recv.at[f_k],
                    device_id=x_peer, device_id_type=pl.DeviceIdType.MESH,
                )
                fw.start()
                x_rdmas[f_k] = fw

        for k in range(C):
            x_rdmas[k].wait_recv()
        for key, st in stores.items():
            if key not in store_waited:
                st.wait()
        for k in range(C):
            z_rdmas[k].wait_send()
            x_rdmas[k].wait_send()

    return pl.pallas_call(
        body,
        out_shape=jax.ShapeDtypeStruct((2 * m, n), jnp.bfloat16),
        in_specs=[pl.BlockSpec(memory_space=pl.ANY)],
        out_specs=pl.BlockSpec(memory_space=pl.ANY),
        scratch_shapes=[
            pltpu.VMEM((NSLOTS, c, n), jnp.float32),
            pltpu.VMEM((C, c, n), jnp.bfloat16),
            pltpu.VMEM((2, c, n), jnp.bfloat16),
            pltpu.SemaphoreType.DMA((NSLOTS,)),
            pltpu.SemaphoreType.DMA((2 * C,)),
            pltpu.SemaphoreType.DMA((C,)),
            pltpu.SemaphoreType.DMA((C,)),
            pltpu.SemaphoreType.DMA((C,)),
            pltpu.SemaphoreType.DMA((C,)),
        ],
        compiler_params=pltpu.CompilerParams(collective_id=0),
    )(x)


# device time: 226220 ns/iter; 4.8620x vs baseline; 1.0805x over previous
import jax
import jax.numpy as jnp
from jax import lax
from jax.experimental import pallas as pl
from jax.experimental.pallas import tpu as pltpu

CQ = 8
NSLOTS = 6
HALF = CQ // 2

_PLAN = [
    ([("s", 0), ("s", 1)], None),
    ([("s", 2)], ("z", 0)),
    ([("s", 3)], ("z", 1)),
    ([("s", 4), ("k", 0)], ("xd", 0)),
    ([("s", 5), ("k", 1)], ("z", 2)),
    ([("s", 6), ("k", 2)], ("xd", 1)),
    ([("s", 7), ("k", 3)], ("z", 3)),
    ([("k", 4), ("k", 5)], ("xd", 2)),
    ([("k", 6), ("k", 7)], ("z", 4)),
    ([("k", 8), ("k", 9)], ("xd", 3)),
    ([("k", 10), ("k", 11)], ("z", 5)),
    ([("k", 12), ("k", 13)], ("yd", 4)),
    ([("k", 14), ("k", 15)], ("z", 6)),
    ([("k", 16), ("k", 17)], ("yd", 5)),
    ([("k", 18), ("k", 19)], ("z", 7)),
    ([("k", 20), ("k", 21)], ("yd", 6)),
    ([("k", 22), ("k", 23)], ("yd", 7)),
]


def kernel(x):
    m, n = x.shape
    qh = m // 4
    c = qh // CQ

    def body(x_ref, out_ref, vx, vc, vk, load_sems, store_sems,
             z_send, z_recv, xd_send, xd_recv, yd_send, yd_recv,
             xg_send, xg_recv, yg_send, yg_recv):
        my_x = lax.axis_index("x")
        my_y = lax.axis_index("y")
        my_z = lax.axis_index("z")
        z_peer = (my_x, my_y, 1 - my_z)
        x_peer = (1 - my_x, my_y, my_z)
        y_peer = (my_x, 1 - my_y, my_z)

        barrier_sem = pltpu.get_barrier_semaphore()
        for peer in (z_peer, x_peer, y_peer):
            pl.semaphore_signal(
                barrier_sem, inc=1, device_id=peer,
                device_id_type=pl.DeviceIdType.MESH,
            )
        pl.semaphore_wait(barrier_sem, 3)

        my_base = my_z * m
        oth_base = (1 - my_z) * m
        q_mine = (2 * my_x + my_y) * qh
        q_x = (2 * (1 - my_x) + my_y) * qh
        q_y = (2 * my_x + (1 - my_y)) * qh
        q_d = (2 * (1 - my_x) + (1 - my_y)) * qh

        def rdma(row, send_sem, recv_sem, peer, src=None):
            dst = out_ref.at[pl.ds(row, c), :]
            return pltpu.make_async_remote_copy(
                src_ref=dst if src is None else src,
                dst_ref=dst, send_sem=send_sem, recv_sem=recv_sem,
                device_id=peer, device_id_type=pl.DeviceIdType.MESH,
            )

        xd_in = [rdma(oth_base + q_x + i * c, xd_send.at[i],
                      xd_recv.at[i], x_peer) for i in range(CQ)]
        yd_in = [rdma(oth_base + q_y + i * c, yd_send.at[i],
                      yd_recv.at[i], y_peer) for i in range(CQ)]
        xg_in = [rdma(oth_base + q_d + (j + HALF) * c, xg_send.at[j],
                      xg_recv.at[j], x_peer) for j in range(HALF)]
        yg_in = [rdma(oth_base + q_d + j * c, yg_send.at[j],
                      yg_recv.at[j], y_peer) for j in range(HALF)]

        keep_offs = [q_x, q_y, q_d]
        casts = [ci for cast_items, _ in _PLAN for ci in cast_items]
        lmap = {}
        loads = []
        for li, (kind, idx) in enumerate(casts):
            if kind == "s":
                off = q_mine + idx * c
            else:
                off = keep_offs[idx // CQ] + (idx % CQ) * c
            loads.append(
                pltpu.make_async_copy(
                    x_ref.at[pl.ds(off, c), :], vx.at[li % NSLOTS],
                    load_sems.at[li % NSLOTS],
                )
            )
            lmap[(kind, idx)] = (li, off)

        state = {"started": 0, "casted": 0}

        def pump_loads():
            while (state["started"] < len(loads)
                   and state["started"] - state["casted"] < NSLOTS - 1):
                loads[state["started"]].start()
                state["started"] += 1

        stores = {}
        store_waited = set()
        z_rdmas = [None] * CQ
        fwds = []

        def do_cast(kind, idx):
            li, off = lmap[(kind, idx)]
            loads[li].wait()
            state["casted"] += 1
            pump_loads()
            if kind == "s":
                vc[idx] = vx[li % NSLOTS].astype(jnp.bfloat16)
                buf = vc.at[idx]
                sidx = idx
            else:
                if idx >= 2 and ("k", idx - 2) not in store_waited:
                    stores[("k", idx - 2)].wait()
                    store_waited.add(("k", idx - 2))
                vk[idx % 2] = vx[li % NSLOTS].astype(jnp.bfloat16)
                buf = vk.at[idx % 2]
                sidx = CQ + idx
            st = pltpu.make_async_copy(
                buf, out_ref.at[pl.ds(my_base + off, c), :],
                store_sems.at[sidx],
            )
            st.start()
            stores[(kind, idx)] = st
            if kind == "s":
                rd = rdma(my_base + off, z_send.at[idx], z_recv.at[idx],
                          z_peer, src=vc.at[idx])
                rd.start()
                z_rdmas[idx] = rd

        def do_wait(stream, i):
            if stream == "z":
                z_rdmas[i].wait_recv()
                row = oth_base + q_mine + i * c
                for sem_s, sem_r, peer in (
                    (xd_send.at[i], xd_recv.at[i], x_peer),
                    (yd_send.at[i], yd_recv.at[i], y_peer),
                ):
                    fw = rdma(row, sem_s, sem_r, peer)
                    fw.start()
                    fwds.append(fw)
            elif stream == "xd":
                xd_in[i].wait_recv()
                if i < HALF:
                    fw = rdma(oth_base + q_x + i * c, yg_send.at[i],
                              yg_recv.at[i], y_peer)
                    fw.start()
                    fwds.append(fw)
            else:
                yd_in[i].wait_recv()
                if i >= HALF:
                    j = i - HALF
                    fw = rdma(oth_base + q_y + i * c, xg_send.at[j],
                              xg_recv.at[j], x_peer)
                    fw.start()
                    fwds.append(fw)

        pump_loads()
        for cast_items, wait_item in _PLAN:
            for kind, idx in cast_items:
                do_cast(kind, idx)
            if wait_item is not None:
                do_wait(*wait_item)

        for i in range(HALF, CQ):
            xd_in[i].wait_recv()
        for i in range(HALF):
            yd_in[i].wait_recv()
        for j in range(HALF):
            xg_in[j].wait_recv()
            yg_in[j].wait_recv()
        for key, st in stores.items():
            if key not in store_waited:
                st.wait()
        for rd in z_rdmas:
            rd.wait_send()
        for fw in fwds:
            fw.wait_send()

    return pl.pallas_call(
        body,
        out_shape=jax.ShapeDtypeStruct((2 * m, n), jnp.bfloat16),
        in_specs=[pl.BlockSpec(memory_space=pl.ANY)],
        out_specs=pl.BlockSpec(memory_space=pl.ANY),
        scratch_shapes=[
            pltpu.VMEM((NSLOTS, c, n), jnp.float32),
            pltpu.VMEM((CQ, c, n), jnp.bfloat16),
            pltpu.VMEM((2, c, n), jnp.bfloat16),
            pltpu.SemaphoreType.DMA((NSLOTS,)),
            pltpu.SemaphoreType.DMA((4 * CQ,)),
            pltpu.SemaphoreType.DMA((CQ,)),
            pltpu.SemaphoreType.DMA((CQ,)),
            pltpu.SemaphoreType.DMA((CQ,)),
            pltpu.SemaphoreType.DMA((CQ,)),
            pltpu.SemaphoreType.DMA((CQ,)),
            pltpu.SemaphoreType.DMA((CQ,)),
            pltpu.SemaphoreType.DMA((HALF,)),
            pltpu.SemaphoreType.DMA((HALF,)),
            pltpu.SemaphoreType.DMA((HALF,)),
            pltpu.SemaphoreType.DMA((HALF,)),
        ],
        compiler_params=pltpu.CompilerParams(collective_id=0),
    )(x)


# device time: 202424 ns/iter; 5.4335x vs baseline; 1.1176x over previous
import jax
import jax.numpy as jnp
from jax import lax
from jax.experimental import pallas as pl
from jax.experimental.pallas import tpu as pltpu

CQ = 8
NSLOTS = 6
HALF = CQ // 2

_PLAN = [
    ([("s", 0), ("s", 1)], None),
    ([("s", 2)], ("z", 0)),
    ([("s", 3)], ("z", 1)),
    ([("s", 4), ("k", 0)], ("xd", 0)),
    ([("s", 5), ("k", 1)], ("z", 2)),
    ([("s", 6), ("k", 2)], ("yd", 1)),
    ([("s", 7), ("k", 3)], ("z", 3)),
    ([("k", 4), ("k", 5)], ("xd", 2)),
    ([("k", 6), ("k", 7)], ("z", 4)),
    ([("k", 8), ("k", 9)], ("yd", 3)),
    ([("k", 10), ("k", 11)], ("z", 5)),
    ([("k", 12), ("k", 13)], ("xd", 4)),
    ([("k", 14), ("k", 15)], ("z", 6)),
    ([("k", 16), ("k", 17)], ("yd", 5)),
    ([("k", 18), ("k", 19)], ("z", 7)),
    ([("k", 20), ("k", 21)], ("xd", 6)),
    ([("k", 22), ("k", 23)], ("yd", 7)),
]


def kernel(x):
    m, n = x.shape
    qh = m // 4
    c = qh // CQ

    def body(x_ref, out_ref, vx, vc, vk, load_sems, store_sems,
             z_send, z_recv, xd_send, xd_recv, yd_send, yd_recv,
             xg_send, xg_recv, yg_send, yg_recv):
        my_x = lax.axis_index("x")
        my_y = lax.axis_index("y")
        my_z = lax.axis_index("z")
        z_peer = (my_x, my_y, 1 - my_z)
        x_peer = (1 - my_x, my_y, my_z)
        y_peer = (my_x, 1 - my_y, my_z)

        barrier_sem = pltpu.get_barrier_semaphore()
        for peer in (z_peer, x_peer, y_peer):
            pl.semaphore_signal(
                barrier_sem, inc=1, device_id=peer,
                device_id_type=pl.DeviceIdType.MESH,
            )
        pl.semaphore_wait(barrier_sem, 3)

        my_base = my_z * m
        oth_base = (1 - my_z) * m
        q_mine = (2 * my_x + my_y) * qh
        q_x = (2 * (1 - my_x) + my_y) * qh
        q_y = (2 * my_x + (1 - my_y)) * qh
        q_d = (2 * (1 - my_x) + (1 - my_y)) * qh

        def rdma(row, send_sem, recv_sem, peer, src=None):
            dst = out_ref.at[pl.ds(row, c), :]
            return pltpu.make_async_remote_copy(
                src_ref=dst if src is None else src,
                dst_ref=dst, send_sem=send_sem, recv_sem=recv_sem,
                device_id=peer, device_id_type=pl.DeviceIdType.MESH,
            )

        xd_in = [rdma(oth_base + q_x + i * c, xd_send.at[i],
                      xd_recv.at[i], x_peer) for i in range(CQ)]
        yd_in = [rdma(oth_base + q_y + i * c, yd_send.at[i],
                      yd_recv.at[i], y_peer) for i in range(CQ)]
        xg_in = [rdma(oth_base + q_d + (2 * j + 1) * c, xg_send.at[j],
                      xg_recv.at[j], x_peer) for j in range(HALF)]
        yg_in = [rdma(oth_base + q_d + (2 * j) * c, yg_send.at[j],
                      yg_recv.at[j], y_peer) for j in range(HALF)]

        keep_offs = [q_x, q_y, q_d]
        casts = [ci for cast_items, _ in _PLAN for ci in cast_items]
        lmap = {}
        loads = []
        for li, (kind, idx) in enumerate(casts):
            if kind == "s":
                off = q_mine + idx * c
            else:
                off = keep_offs[idx // CQ] + (idx % CQ) * c
            loads.append(
                pltpu.make_async_copy(
                    x_ref.at[pl.ds(off, c), :], vx.at[li % NSLOTS],
                    load_sems.at[li % NSLOTS],
                )
            )
            lmap[(kind, idx)] = (li, off)

        state = {"started": 0, "casted": 0}

        def pump_loads():
            while (state["started"] < len(loads)
                   and state["started"] - state["casted"] < NSLOTS - 1):
                loads[state["started"]].start()
                state["started"] += 1

        stores = {}
        store_waited = set()
        z_rdmas = [None] * CQ
        fwds = []

        def do_cast(kind, idx):
            li, off = lmap[(kind, idx)]
            loads[li].wait()
            state["casted"] += 1
            pump_loads()
            if kind == "s":
                vc[idx] = vx[li % NSLOTS].astype(jnp.bfloat16)
                buf = vc.at[idx]
                sidx = idx
            else:
                if idx >= 2 and ("k", idx - 2) not in store_waited:
                    stores[("k", idx - 2)].wait()
                    store_waited.add(("k", idx - 2))
                vk[idx % 2] = vx[li % NSLOTS].astype(jnp.bfloat16)
                buf = vk.at[idx % 2]
                sidx = CQ + idx
            st = pltpu.make_async_copy(
                buf, out_ref.at[pl.ds(my_base + off, c), :],
                store_sems.at[sidx],
            )
            st.start()
            stores[(kind, idx)] = st
            if kind == "s":
                rd = rdma(my_base + off, z_send.at[idx], z_recv.at[idx],
                          z_peer, src=vc.at[idx])
                rd.start()
                z_rdmas[idx] = rd

        def do_wait(stream, i):
            if stream == "z":
                z_rdmas[i].wait_recv()
                row = oth_base + q_mine + i * c
                for sem_s, sem_r, peer in (
                    (xd_send.at[i], xd_recv.at[i], x_peer),
                    (yd_send.at[i], yd_recv.at[i], y_peer),
                ):
                    fw = rdma(row, sem_s, sem_r, peer)
                    fw.start()
                    fwds.append(fw)
            elif stream == "xd":
                xd_in[i].wait_recv()
                if i % 2 == 0:
                    fw = rdma(oth_base + q_x + i * c, yg_send.at[i // 2],
                              yg_recv.at[i // 2], y_peer)
                    fw.start()
                    fwds.append(fw)
            else:
                yd_in[i].wait_recv()
                if i % 2 == 1:
                    fw = rdma(oth_base + q_y + i * c, xg_send.at[i // 2],
                              xg_recv.at[i // 2], x_peer)
                    fw.start()
                    fwds.append(fw)

        pump_loads()
        for cast_items, wait_item in _PLAN:
            for kind, idx in cast_items:
                do_cast(kind, idx)
            if wait_item is not None:
                do_wait(*wait_item)

        for i in range(1, CQ, 2):
            xd_in[i].wait_recv()
        for i in range(0, CQ, 2):
            yd_in[i].wait_recv()
        for j in range(HALF):
            xg_in[j].wait_recv()
            yg_in[j].wait_recv()
        for key, st in stores.items():
            if key not in store_waited:
                st.wait()
        for rd in z_rdmas:
            rd.wait_send()
        for fw in fwds:
            fw.wait_send()

    return pl.pallas_call(
        body,
        out_shape=jax.ShapeDtypeStruct((2 * m, n), jnp.bfloat16),
        in_specs=[pl.BlockSpec(memory_space=pl.ANY)],
        out_specs=pl.BlockSpec(memory_space=pl.ANY),
        scratch_shapes=[
            pltpu.VMEM((NSLOTS, c, n), jnp.float32),
            pltpu.VMEM((CQ, c, n), jnp.bfloat16),
            pltpu.VMEM((2, c, n), jnp.bfloat16),
            pltpu.SemaphoreType.DMA((NSLOTS,)),
            pltpu.SemaphoreType.DMA((4 * CQ,)),
            pltpu.SemaphoreType.DMA((CQ,)),
            pltpu.SemaphoreType.DMA((CQ,)),
            pltpu.SemaphoreType.DMA((CQ,)),
            pltpu.SemaphoreType.DMA((CQ,)),
            pltpu.SemaphoreType.DMA((CQ,)),
            pltpu.SemaphoreType.DMA((CQ,)),
            pltpu.SemaphoreType.DMA((HALF,)),
            pltpu.SemaphoreType.DMA((HALF,)),
            pltpu.SemaphoreType.DMA((HALF,)),
            pltpu.SemaphoreType.DMA((HALF,)),
        ],
        compiler_params=pltpu.CompilerParams(collective_id=0),
    )(x)


# device time: 196936 ns/iter; 5.5849x vs baseline; 1.0279x over previous
import jax
import jax.numpy as jnp
from jax import lax
from jax.experimental import pallas as pl
from jax.experimental.pallas import tpu as pltpu

CQ = 16
NSLOTS = 8
HALF = CQ // 2


def _make_plan(cq):
    seq = [("z", 0), ("z", 1)]
    zi = 2
    for i in range(cq):
        seq.append(("xd", i) if i % 2 == 0 else ("yd", i))
        if zi < cq:
            seq.append(("z", zi))
            zi += 1
    casts = [("s", i) for i in range(2, cq)] + [
        ("k", t) for t in range(3 * cq)
    ]
    plan = [([("s", 0), ("s", 1)], None)]
    idx = 0
    for w in seq:
        plan.append((casts[idx:idx + 2], w))
        idx += 2
    assert idx >= len(casts)
    return plan


_PLAN = _make_plan(CQ)


def kernel(x):
    m, n = x.shape
    qh = m // 4
    c = qh // CQ

    def body(x_ref, out_ref, vx, vc, vk, load_sems, store_sems,
             z_send, z_recv, xd_send, xd_recv, yd_send, yd_recv,
             xg_send, xg_recv, yg_send, yg_recv):
        my_x = lax.axis_index("x")
        my_y = lax.axis_index("y")
        my_z = lax.axis_index("z")
        z_peer = (my_x, my_y, 1 - my_z)
        x_peer = (1 - my_x, my_y, my_z)
        y_peer = (my_x, 1 - my_y, my_z)

        barrier_sem = pltpu.get_barrier_semaphore()
        for peer in (z_peer, x_peer, y_peer):
            pl.semaphore_signal(
                barrier_sem, inc=1, device_id=peer,
                device_id_type=pl.DeviceIdType.MESH,
            )
        pl.semaphore_wait(barrier_sem, 3)

        my_base = my_z * m
        oth_base = (1 - my_z) * m
        q_mine = (2 * my_x + my_y) * qh
        q_x = (2 * (1 - my_x) + my_y) * qh
        q_y = (2 * my_x + (1 - my_y)) * qh
        q_d = (2 * (1 - my_x) + (1 - my_y)) * qh

        def rdma(row, send_sem, recv_sem, peer, src=None):
            dst = out_ref.at[pl.ds(row, c), :]
            return pltpu.make_async_remote_copy(
                src_ref=dst if src is None else src,
                dst_ref=dst, send_sem=send_sem, recv_sem=recv_sem,
                device_id=peer, device_id_type=pl.DeviceIdType.MESH,
            )

        xd_in = [rdma(oth_base + q_x + i * c, xd_send.at[i],
                      xd_recv.at[i], x_peer) for i in range(CQ)]
        yd_in = [rdma(oth_base + q_y + i * c, yd_send.at[i],
                      yd_recv.at[i], y_peer) for i in range(CQ)]
        xg_in = [rdma(oth_base + q_d + (2 * j + 1) * c, xg_send.at[j],
                      xg_recv.at[j], x_peer) for j in range(HALF)]
        yg_in = [rdma(oth_base + q_d + (2 * j) * c, yg_send.at[j],
                      yg_recv.at[j], y_peer) for j in range(HALF)]

        keep_offs = [q_x, q_y, q_d]
        casts = [ci for cast_items, _ in _PLAN for ci in cast_items]
        lmap = {}
        loads = []
        for li, (kind, idx) in enumerate(casts):
            if kind == "s":
                off = q_mine + idx * c
            else:
                off = keep_offs[idx // CQ] + (idx % CQ) * c
            loads.append(
                pltpu.make_async_copy(
                    x_ref.at[pl.ds(off, c), :], vx.at[li % NSLOTS],
                    load_sems.at[li % NSLOTS],
                )
            )
            lmap[(kind, idx)] = (li, off)

        state = {"started": 0, "casted": 0}

        def pump_loads():
            while (state["started"] < len(loads)
                   and state["started"] - state["casted"] < NSLOTS - 1):
                loads[state["started"]].start()
                state["started"] += 1

        stores = {}
        store_waited = set()
        z_rdmas = [None] * CQ
        fwds = []

        def do_cast(kind, idx):
            li, off = lmap[(kind, idx)]
            loads[li].wait()
            state["casted"] += 1
            pump_loads()
            if kind == "s":
                vc[idx] = vx[li % NSLOTS].astype(jnp.bfloat16)
                buf = vc.at[idx]
                sidx = idx
            else:
                if idx >= 2 and ("k", idx - 2) not in store_waited:
                    stores[("k", idx - 2)].wait()
                    store_waited.add(("k", idx - 2))
                vk[idx % 2] = vx[li % NSLOTS].astype(jnp.bfloat16)
                buf = vk.at[idx % 2]
                sidx = CQ + idx
            st = pltpu.make_async_copy(
                buf, out_ref.at[pl.ds(my_base + off, c), :],
                store_sems.at[sidx],
            )
            st.start()
            stores[(kind, idx)] = st
            if kind == "s":
                rd = rdma(my_base + off, z_send.at[idx], z_recv.at[idx],
                          z_peer, src=vc.at[idx])
                rd.start()
                z_rdmas[idx] = rd

        def do_wait(stream, i):
            if stream == "z":
                z_rdmas[i].wait_recv()
                row = oth_base + q_mine + i * c
                for sem_s, sem_r, peer in (
                    (xd_send.at[i], xd_recv.at[i], x_peer),
                    (yd_send.at[i], yd_recv.at[i], y_peer),
                ):
                    fw = rdma(row, sem_s, sem_r, peer)
                    fw.start()
                    fwds.append(fw)
            elif stream == "xd":
                xd_in[i].wait_recv()
                if i % 2 == 0:
                    fw = rdma(oth_base + q_x + i * c, yg_send.at[i // 2],
                              yg_recv.at[i // 2], y_peer)
                    fw.start()
                    fwds.append(fw)
            else:
                yd_in[i].wait_recv()
                if i % 2 == 1:
                    fw = rdma(oth_base + q_y + i * c, xg_send.at[i // 2],
                              xg_recv.at[i // 2], x_peer)
                    fw.start()
                    fwds.append(fw)

        pump_loads()
        for cast_items, wait_item in _PLAN:
            for kind, idx in cast_items:
                do_cast(kind, idx)
            if wait_item is not None:
                do_wait(*wait_item)

        for i in range(1, CQ, 2):
            xd_in[i].wait_recv()
        for i in range(0, CQ, 2):
            yd_in[i].wait_recv()
        for j in range(HALF):
            xg_in[j].wait_recv()
            yg_in[j].wait_recv()
        for key, st in stores.items():
            if key not in store_waited:
                st.wait()
        for rd in z_rdmas:
            rd.wait_send()
        for fw in fwds:
            fw.wait_send()

    return pl.pallas_call(
        body,
        out_shape=jax.ShapeDtypeStruct((2 * m, n), jnp.bfloat16),
        in_specs=[pl.BlockSpec(memory_space=pl.ANY)],
        out_specs=pl.BlockSpec(memory_space=pl.ANY),
        scratch_shapes=[
            pltpu.VMEM((NSLOTS, c, n), jnp.float32),
            pltpu.VMEM((CQ, c, n), jnp.bfloat16),
            pltpu.VMEM((2, c, n), jnp.bfloat16),
            pltpu.SemaphoreType.DMA((NSLOTS,)),
            pltpu.SemaphoreType.DMA((4 * CQ,)),
            pltpu.SemaphoreType.DMA((CQ,)),
            pltpu.SemaphoreType.DMA((CQ,)),
            pltpu.SemaphoreType.DMA((CQ,)),
            pltpu.SemaphoreType.DMA((CQ,)),
            pltpu.SemaphoreType.DMA((CQ,)),
            pltpu.SemaphoreType.DMA((CQ,)),
            pltpu.SemaphoreType.DMA((HALF,)),
            pltpu.SemaphoreType.DMA((HALF,)),
            pltpu.SemaphoreType.DMA((HALF,)),
            pltpu.SemaphoreType.DMA((HALF,)),
        ],
        compiler_params=pltpu.CompilerParams(collective_id=0),
    )(x)


# device time: 194677 ns/iter; 5.6497x vs baseline; 1.0116x over previous
import jax
import jax.numpy as jnp
from jax import lax
from jax.experimental import pallas as pl
from jax.experimental.pallas import tpu as pltpu

CQ = 32
NSLOTS = 8
HALF = CQ // 2


def _make_plan(cq):
    seq = [("z", 0), ("z", 1)]
    zi = 2
    for i in range(cq):
        seq.append(("xd", i) if i % 2 == 0 else ("yd", i))
        if zi < cq:
            seq.append(("z", zi))
            zi += 1
    casts = [("s", i) for i in range(2, cq)] + [
        ("k", t) for t in range(3 * cq)
    ]
    plan = [([("s", 0), ("s", 1)], None)]
    idx = 0
    for w in seq:
        plan.append((casts[idx:idx + 2], w))
        idx += 2
    assert idx >= len(casts)
    return plan


_PLAN = _make_plan(CQ)


def kernel(x):
    m, n = x.shape
    qh = m // 4
    c = qh // CQ

    def body(x_ref, out_ref, vx, vc, vk, load_sems, store_sems,
             z_send, z_recv, xd_send, xd_recv, yd_send, yd_recv,
             xg_send, xg_recv, yg_send, yg_recv):
        my_x = lax.axis_index("x")
        my_y = lax.axis_index("y")
        my_z = lax.axis_index("z")
        z_peer = (my_x, my_y, 1 - my_z)
        x_peer = (1 - my_x, my_y, my_z)
        y_peer = (my_x, 1 - my_y, my_z)

        barrier_sem = pltpu.get_barrier_semaphore()
        for peer in (z_peer, x_peer, y_peer):
            pl.semaphore_signal(
                barrier_sem, inc=1, device_id=peer,
                device_id_type=pl.DeviceIdType.MESH,
            )
        pl.semaphore_wait(barrier_sem, 3)

        my_base = my_z * m
        oth_base = (1 - my_z) * m
        q_mine = (2 * my_x + my_y) * qh
        q_x = (2 * (1 - my_x) + my_y) * qh
        q_y = (2 * my_x + (1 - my_y)) * qh
        q_d = (2 * (1 - my_x) + (1 - my_y)) * qh

        def rdma(row, send_sem, recv_sem, peer, src=None):
            dst = out_ref.at[pl.ds(row, c), :]
            return pltpu.make_async_remote_copy(
                src_ref=dst if src is None else src,
                dst_ref=dst, send_sem=send_sem, recv_sem=recv_sem,
                device_id=peer, device_id_type=pl.DeviceIdType.MESH,
            )

        xd_in = [rdma(oth_base + q_x + i * c, xd_send.at[i],
                      xd_recv.at[i], x_peer) for i in range(CQ)]
        yd_in = [rdma(oth_base + q_y + i * c, yd_send.at[i],
                      yd_recv.at[i], y_peer) for i in range(CQ)]
        xg_in = [rdma(oth_base + q_d + (2 * j + 1) * c, xg_send.at[j],
                      xg_recv.at[j], x_peer) for j in range(HALF)]
        yg_in = [rdma(oth_base + q_d + (2 * j) * c, yg_send.at[j],
                      yg_recv.at[j], y_peer) for j in range(HALF)]

        keep_offs = [q_x, q_y, q_d]
        casts = [ci for cast_items, _ in _PLAN for ci in cast_items]
        lmap = {}
        loads = []
        for li, (kind, idx) in enumerate(casts):
            if kind == "s":
                off = q_mine + idx * c
            else:
                off = keep_offs[idx // CQ] + (idx % CQ) * c
            loads.append(
                pltpu.make_async_copy(
                    x_ref.at[pl.ds(off, c), :], vx.at[li % NSLOTS],
                    load_sems.at[li % NSLOTS],
                )
            )
            lmap[(kind, idx)] = (li, off)

        state = {"started": 0, "casted": 0}

        def pump_loads():
            while (state["started"] < len(loads)
                   and state["started"] - state["casted"] < NSLOTS - 1):
                loads[state["started"]].start()
                state["started"] += 1

        stores = {}
        store_waited = set()
        z_rdmas = [None] * CQ
        fwds = []

        def do_cast(kind, idx):
            li, off = lmap[(kind, idx)]
            loads[li].wait()
            state["casted"] += 1
            pump_loads()
            if kind == "s":
                vc[idx] = vx[li % NSLOTS].astype(jnp.bfloat16)
                buf = vc.at[idx]
                sidx = idx
            else:
                if idx >= 2 and ("k", idx - 2) not in store_waited:
                    stores[("k", idx - 2)].wait()
                    store_waited.add(("k", idx - 2))
                vk[idx % 2] = vx[li % NSLOTS].astype(jnp.bfloat16)
                buf = vk.at[idx % 2]
                sidx = CQ + idx
            st = pltpu.make_async_copy(
                buf, out_ref.at[pl.ds(my_base + off, c), :],
                store_sems.at[sidx],
            )
            st.start()
            stores[(kind, idx)] = st
            if kind == "s":
                rd = rdma(my_base + off, z_send.at[idx], z_recv.at[idx],
                          z_peer, src=vc.at[idx])
                rd.start()
                z_rdmas[idx] = rd

        def do_wait(stream, i):
            if stream == "z":
                z_rdmas[i].wait_recv()
                row = oth_base + q_mine + i * c
                for sem_s, sem_r, peer in (
                    (xd_send.at[i], xd_recv.at[i], x_peer),
                    (yd_send.at[i], yd_recv.at[i], y_peer),
                ):
                    fw = rdma(row, sem_s, sem_r, peer)
                    fw.start()
                    fwds.append(fw)
            elif stream == "xd":
                xd_in[i].wait_recv()
                if i % 2 == 0:
                    fw = rdma(oth_base + q_x + i * c, yg_send.at[i // 2],
                              yg_recv.at[i // 2], y_peer)
                    fw.start()
                    fwds.append(fw)
            else:
                yd_in[i].wait_recv()
                if i % 2 == 1:
                    fw = rdma(oth_base + q_y + i * c, xg_send.at[i // 2],
                              xg_recv.at[i // 2], x_peer)
                    fw.start()
                    fwds.append(fw)

        pump_loads()
        for cast_items, wait_item in _PLAN:
            for kind, idx in cast_items:
                do_cast(kind, idx)
            if wait_item is not None:
                do_wait(*wait_item)

        for i in range(1, CQ, 2):
            xd_in[i].wait_recv()
        for i in range(0, CQ, 2):
            yd_in[i].wait_recv()
        for j in range(HALF):
            xg_in[j].wait_recv()
            yg_in[j].wait_recv()
        for key, st in stores.items():
            if key not in store_waited:
                st.wait()
        for rd in z_rdmas:
            rd.wait_send()
        for fw in fwds:
            fw.wait_send()

    return pl.pallas_call(
        body,
        out_shape=jax.ShapeDtypeStruct((2 * m, n), jnp.bfloat16),
        in_specs=[pl.BlockSpec(memory_space=pl.ANY)],
        out_specs=pl.BlockSpec(memory_space=pl.ANY),
        scratch_shapes=[
            pltpu.VMEM((NSLOTS, c, n), jnp.float32),
            pltpu.VMEM((CQ, c, n), jnp.bfloat16),
            pltpu.VMEM((2, c, n), jnp.bfloat16),
            pltpu.SemaphoreType.DMA((NSLOTS,)),
            pltpu.SemaphoreType.DMA((4 * CQ,)),
            pltpu.SemaphoreType.DMA((CQ,)),
            pltpu.SemaphoreType.DMA((CQ,)),
            pltpu.SemaphoreType.DMA((CQ,)),
            pltpu.SemaphoreType.DMA((CQ,)),
            pltpu.SemaphoreType.DMA((CQ,)),
            pltpu.SemaphoreType.DMA((CQ,)),
            pltpu.SemaphoreType.DMA((HALF,)),
            pltpu.SemaphoreType.DMA((HALF,)),
            pltpu.SemaphoreType.DMA((HALF,)),
            pltpu.SemaphoreType.DMA((HALF,)),
        ],
        compiler_params=pltpu.CompilerParams(collective_id=0),
    )(x)


# device time: 181774 ns/iter; 6.0508x vs baseline; 1.0710x over previous
import jax
import jax.numpy as jnp
from jax import lax
from jax.experimental import pallas as pl
from jax.experimental.pallas import tpu as pltpu

CQ = 32
NSLOTS = 8
HALF = CQ // 2

DIAGZ = [i for i in range(CQ) if i % 3 == 2]
DIAG_X = [i for i in range(CQ) if i % 3 != 2 and i % 2 == 1]
DIAG_Y = [i for i in range(CQ) if i % 3 != 2 and i % 2 == 0]
_XPOS = {i: j for j, i in enumerate(DIAG_X)}
_YPOS = {i: j for j, i in enumerate(DIAG_Y)}
_ZPOS = {i: j for j, i in enumerate(DIAGZ)}


def _make_plan(cq):
    seq = [("z", 0), ("z", 1)]
    zi = 2
    for i in range(cq):
        seq.append(("xd", i) if i % 2 == 0 else ("yd", i))
        if zi < cq:
            seq.append(("z", zi))
            zi += 1
    casts = [("s", i) for i in range(2, cq)] + [
        ("k", t) for t in range(3 * cq)
    ]
    plan = [([("s", 0), ("s", 1)], None)]
    idx = 0
    for w in seq:
        plan.append((casts[idx:idx + 2], w))
        idx += 2
    assert idx >= len(casts)
    return plan


_PLAN = _make_plan(CQ)


def kernel(x):
    m, n = x.shape
    qh = m // 4
    c = qh // CQ

    def body(x_ref, out_ref, vx, vc, vk, load_sems, store_sems,
             z_send, z_recv, xd_send, xd_recv, yd_send, yd_recv,
             xg_send, xg_recv, yg_send, yg_recv, zq_send, zq_recv):
        my_x = lax.axis_index("x")
        my_y = lax.axis_index("y")
        my_z = lax.axis_index("z")
        z_peer = (my_x, my_y, 1 - my_z)
        x_peer = (1 - my_x, my_y, my_z)
        y_peer = (my_x, 1 - my_y, my_z)

        barrier_sem = pltpu.get_barrier_semaphore()
        for peer in (z_peer, x_peer, y_peer):
            pl.semaphore_signal(
                barrier_sem, inc=1, device_id=peer,
                device_id_type=pl.DeviceIdType.MESH,
            )
        pl.semaphore_wait(barrier_sem, 3)

        my_base = my_z * m
        oth_base = (1 - my_z) * m
        q_mine = (2 * my_x + my_y) * qh
        q_x = (2 * (1 - my_x) + my_y) * qh
        q_y = (2 * my_x + (1 - my_y)) * qh
        q_d = (2 * (1 - my_x) + (1 - my_y)) * qh

        def rdma(row, send_sem, recv_sem, peer, src=None):
            dst = out_ref.at[pl.ds(row, c), :]
            return pltpu.make_async_remote_copy(
                src_ref=dst if src is None else src,
                dst_ref=dst, send_sem=send_sem, recv_sem=recv_sem,
                device_id=peer, device_id_type=pl.DeviceIdType.MESH,
            )

        xd_in = [rdma(oth_base + q_x + i * c, xd_send.at[i],
                      xd_recv.at[i], x_peer) for i in range(CQ)]
        yd_in = [rdma(oth_base + q_y + i * c, yd_send.at[i],
                      yd_recv.at[i], y_peer) for i in range(CQ)]
        xg_in = [rdma(oth_base + q_d + i * c, xg_send.at[j],
                      xg_recv.at[j], x_peer) for j, i in enumerate(DIAG_X)]
        yg_in = [rdma(oth_base + q_d + i * c, yg_send.at[j],
                      yg_recv.at[j], y_peer) for j, i in enumerate(DIAG_Y)]
        zg_in = [rdma(oth_base + q_d + i * c, zq_send.at[j],
                      zq_recv.at[j], z_peer) for j, i in enumerate(DIAGZ)]

        keep_offs = [q_x, q_y, q_d]
        casts = [ci for cast_items, _ in _PLAN for ci in cast_items]
        lmap = {}
        loads = []
        for li, (kind, idx) in enumerate(casts):
            if kind == "s":
                off = q_mine + idx * c
            else:
                off = keep_offs[idx // CQ] + (idx % CQ) * c
            loads.append(
                pltpu.make_async_copy(
                    x_ref.at[pl.ds(off, c), :], vx.at[li % NSLOTS],
                    load_sems.at[li % NSLOTS],
                )
            )
            lmap[(kind, idx)] = (li, off)

        state = {"started": 0, "casted": 0}

        def pump_loads():
            while (state["started"] < len(loads)
                   and state["started"] - state["casted"] < NSLOTS - 1):
                loads[state["started"]].start()
                state["started"] += 1

        stores = {}
        store_waited = set()
        z_rdmas = [None] * CQ
        fwds = []

        def do_cast(kind, idx):
            li, off = lmap[(kind, idx)]
            loads[li].wait()
            state["casted"] += 1
            pump_loads()
            if kind == "s":
                vc[idx] = vx[li % NSLOTS].astype(jnp.bfloat16)
                buf = vc.at[idx]
                sidx = idx
            else:
                if idx >= 2 and ("k", idx - 2) not in store_waited:
                    stores[("k", idx - 2)].wait()
                    store_waited.add(("k", idx - 2))
                vk[idx % 2] = vx[li % NSLOTS].astype(jnp.bfloat16)
                buf = vk.at[idx % 2]
                sidx = CQ + idx
            st = pltpu.make_async_copy(
                buf, out_ref.at[pl.ds(my_base + off, c), :],
                store_sems.at[sidx],
            )
            st.start()
            stores[(kind, idx)] = st
            if kind == "s":
                rd = rdma(my_base + off, z_send.at[idx], z_recv.at[idx],
                          z_peer, src=vc.at[idx])
                rd.start()
                z_rdmas[idx] = rd
            elif idx >= 2 * CQ and (idx - 2 * CQ) in _ZPOS:
                i = idx - 2 * CQ
                st.wait()
                store_waited.add((kind, idx))
                fw = rdma(my_base + off, zq_send.at[_ZPOS[i]],
                          zq_recv.at[_ZPOS[i]], z_peer)
                fw.start()
                fwds.append(fw)

        def do_wait(stream, i):
            if stream == "z":
                z_rdmas[i].wait_recv()
                row = oth_base + q_mine + i * c
                for sem_s, sem_r, peer in (
                    (xd_send.at[i], xd_recv.at[i], x_peer),
                    (yd_send.at[i], yd_recv.at[i], y_peer),
                ):
                    fw = rdma(row, sem_s, sem_r, peer)
                    fw.start()
                    fwds.append(fw)
            elif stream == "xd":
                xd_in[i].wait_recv()
                if i in _YPOS:
                    fw = rdma(oth_base + q_x + i * c, yg_send.at[_YPOS[i]],
                              yg_recv.at[_YPOS[i]], y_peer)
                    fw.start()
                    fwds.append(fw)
            else:
                yd_in[i].wait_recv()
                if i in _XPOS:
                    fw = rdma(oth_base + q_y + i * c, xg_send.at[_XPOS[i]],
                              xg_recv.at[_XPOS[i]], x_peer)
                    fw.start()
                    fwds.append(fw)

        pump_loads()
        for cast_items, wait_item in _PLAN:
            for kind, idx in cast_items:
                do_cast(kind, idx)
            if wait_item is not None:
                do_wait(*wait_item)

        for i in range(1, CQ, 2):
            xd_in[i].wait_recv()
        for i in range(0, CQ, 2):
            yd_in[i].wait_recv()
        for d in xg_in + yg_in + zg_in:
            d.wait_recv()
        for key, st in stores.items():
            if key not in store_waited:
                st.wait()
        for rd in z_rdmas:
            rd.wait_send()
        for fw in fwds:
            fw.wait_send()

    return pl.pallas_call(
        body,
        out_shape=jax.ShapeDtypeStruct((2 * m, n), jnp.bfloat16),
        in_specs=[pl.BlockSpec(memory_space=pl.ANY)],
        out_specs=pl.BlockSpec(memory_space=pl.ANY),
        scratch_shapes=[
            pltpu.VMEM((NSLOTS, c, n), jnp.float32),
            pltpu.VMEM((CQ, c, n), jnp.bfloat16),
            pltpu.VMEM((2, c, n), jnp.bfloat16),
            pltpu.SemaphoreType.DMA((NSLOTS,)),
            pltpu.SemaphoreType.DMA((4 * CQ,)),
            pltpu.SemaphoreType.DMA((CQ,)),
            pltpu.SemaphoreType.DMA((CQ,)),
            pltpu.SemaphoreType.DMA((CQ,)),
            pltpu.SemaphoreType.DMA((CQ,)),
            pltpu.SemaphoreType.DMA((CQ,)),
            pltpu.SemaphoreType.DMA((CQ,)),
            pltpu.SemaphoreType.DMA((len(DIAG_X),)),
            pltpu.SemaphoreType.DMA((len(DIAG_X),)),
            pltpu.SemaphoreType.DMA((len(DIAG_Y),)),
            pltpu.SemaphoreType.DMA((len(DIAG_Y),)),
            pltpu.SemaphoreType.DMA((len(DIAGZ),)),
            pltpu.SemaphoreType.DMA((len(DIAGZ),)),
        ],
        compiler_params=pltpu.CompilerParams(collective_id=0),
    )(x)
